# initial kernel scaffold (unmeasured)
import jax
import jax.numpy as jnp
from jax import lax
from jax.experimental import pallas as pl
from jax.experimental.pallas import tpu as pltpu


def kernel(
    x,
):
    def body(*refs):
        pass

    out_shape = jax.ShapeDtypeStruct(..., jnp.float32)
    return pl.pallas_call(body, out_shape=out_shape)(...)


# baseline (device time: 62466 ns/iter reference)
import jax
import jax.numpy as jnp
from jax import lax
from jax.experimental import pallas as pl
from jax.experimental.pallas import tpu as pltpu

N_DEV = 16


def kernel(x):
    m, n = x.shape
    chunk = m // N_DEV

    def body(
        x_ref,
        out_ref,
        xb_ref,
        send_ref,
        comm_ref,
        rs_send_sems,
        rs_recv_sems,
        ag_send_sems,
        ag_recv_sems,
    ):
        my = lax.axis_index("i")
        left = lax.rem(my + N_DEV - 1, N_DEV)
        right = lax.rem(my + 1, N_DEV)

        barrier_sem = pltpu.get_barrier_semaphore()
        for nbr in (left, right):
            pl.semaphore_signal(
                barrier_sem,
                inc=1,
                device_id=(nbr,),
                device_id_type=pl.DeviceIdType.MESH,
            )
        pl.semaphore_wait(barrier_sem, 2)

        xb_ref[...] = x_ref[...].astype(jnp.bfloat16)

        for s in range(N_DEV - 1):
            c = lax.rem(my - s + N_DEV, N_DEV)
            local = xb_ref[pl.ds(c * chunk, chunk), :]
            if s == 0:
                send_ref[...] = local
            else:
                send_ref[...] = comm_ref[s - 1] + local
            rdma = pltpu.make_async_remote_copy(
                src_ref=send_ref,
                dst_ref=comm_ref.at[s],
                send_sem=rs_send_sems.at[s],
                recv_sem=rs_recv_sems.at[s],
                device_id=(right,),
                device_id_type=pl.DeviceIdType.MESH,
            )
            rdma.start()
            rdma.wait()

        r0 = lax.rem(my + 1, N_DEV)
        out_ref[pl.ds(r0 * chunk, chunk), :] = (
            comm_ref[N_DEV - 2] + xb_ref[pl.ds(r0 * chunk, chunk), :]
        )

        for h in range(N_DEV - 1):
            o_send = lax.rem(my + 1 - h + 2 * N_DEV, N_DEV)
            rdma = pltpu.make_async_remote_copy(
                src_ref=out_ref.at[pl.ds(o_send * chunk, chunk), :],
                dst_ref=out_ref.at[pl.ds(o_send * chunk, chunk), :],
                send_sem=ag_send_sems.at[h],
                recv_sem=ag_recv_sems.at[h],
                device_id=(right,),
                device_id_type=pl.DeviceIdType.MESH,
            )
            rdma.start()
            rdma.wait()

    return pl.pallas_call(
        body,
        out_shape=jax.ShapeDtypeStruct((m, n), jnp.bfloat16),
        in_specs=[pl.BlockSpec(memory_space=pltpu.VMEM)],
        out_specs=pl.BlockSpec(memory_space=pltpu.VMEM),
        scratch_shapes=[
            pltpu.VMEM((m, n), jnp.bfloat16),
            pltpu.VMEM((chunk, n), jnp.bfloat16),
            pltpu.VMEM((N_DEV - 1, chunk, n), jnp.bfloat16),
            pltpu.SemaphoreType.DMA((N_DEV - 1,)),
            pltpu.SemaphoreType.DMA((N_DEV - 1,)),
            pltpu.SemaphoreType.DMA((N_DEV - 1,)),
            pltpu.SemaphoreType.DMA((N_DEV - 1,)),
        ],
        compiler_params=pltpu.CompilerParams(collective_id=0),
    )(x)


# device time: 19623 ns/iter; 3.1833x vs baseline; 3.1833x over previous
import jax
import jax.numpy as jnp
from jax import lax
from jax.experimental import pallas as pl
from jax.experimental.pallas import tpu as pltpu

N_DEV = 16
N_ROUNDS = 4


def kernel(x):
    m, n = x.shape

    def body(x_ref, out_ref, comm_ref, send_sems, recv_sems):
        my = lax.axis_index("i")

        barrier_sem = pltpu.get_barrier_semaphore()
        for k in range(N_ROUNDS):
            partner = lax.bitwise_xor(my, 1 << k)
            pl.semaphore_signal(
                barrier_sem,
                inc=1,
                device_id=(partner,),
                device_id_type=pl.DeviceIdType.MESH,
            )
        pl.semaphore_wait(barrier_sem, N_ROUNDS)

        out_ref[...] = x_ref[...].astype(jnp.bfloat16)

        for k in range(N_ROUNDS):
            partner = lax.bitwise_xor(my, 1 << k)
            rdma = pltpu.make_async_remote_copy(
                src_ref=out_ref,
                dst_ref=comm_ref.at[k],
                send_sem=send_sems.at[k],
                recv_sem=recv_sems.at[k],
                device_id=(partner,),
                device_id_type=pl.DeviceIdType.MESH,
            )
            rdma.start()
            rdma.wait()
            out_ref[...] = out_ref[...] + comm_ref[k]

    return pl.pallas_call(
        body,
        out_shape=jax.ShapeDtypeStruct((m, n), jnp.bfloat16),
        in_specs=[pl.BlockSpec(memory_space=pltpu.VMEM)],
        out_specs=pl.BlockSpec(memory_space=pltpu.VMEM),
        scratch_shapes=[
            pltpu.VMEM((N_ROUNDS, m, n), jnp.bfloat16),
            pltpu.SemaphoreType.DMA((N_ROUNDS,)),
            pltpu.SemaphoreType.DMA((N_ROUNDS,)),
        ],
        compiler_params=pltpu.CompilerParams(collective_id=0),
    )(x)


# device time: 17959 ns/iter; 3.4783x vs baseline; 1.0927x over previous
import jax
import jax.numpy as jnp
from jax import lax
from jax.experimental import pallas as pl
from jax.experimental.pallas import tpu as pltpu

N_DEV = 16
ROUND_XOR = (1, 3, 4, 8)
N_ROUNDS = len(ROUND_XOR)
P = 2


def kernel(x):
    m, n = x.shape
    rows = m // P

    def body(x_ref, out_ref, comm_ref, send_sems, recv_sems):
        my = lax.axis_index("i")

        barrier_sem = pltpu.get_barrier_semaphore()
        for xr in ROUND_XOR:
            pl.semaphore_signal(
                barrier_sem,
                inc=1,
                device_id=(lax.bitwise_xor(my, xr),),
                device_id_type=pl.DeviceIdType.MESH,
            )
        pl.semaphore_wait(barrier_sem, N_ROUNDS)

        out_ref[...] = x_ref[...].astype(jnp.bfloat16)

        def make(k, p):
            partner = lax.bitwise_xor(my, ROUND_XOR[k])
            return pltpu.make_async_remote_copy(
                src_ref=out_ref.at[pl.ds(p * rows, rows), :],
                dst_ref=comm_ref.at[k, pl.ds(p * rows, rows), :],
                send_sem=send_sems.at[k, p],
                recv_sem=recv_sems.at[k, p],
                device_id=(partner,),
                device_id_type=pl.DeviceIdType.MESH,
            )

        rdmas = {}
        for p in range(P):
            rdmas[(0, p)] = make(0, p)
            rdmas[(0, p)].start()
        for k in range(N_ROUNDS):
            for p in range(P):
                rdmas[(k, p)].wait()
                sl = pl.ds(p * rows, rows)
                out_ref[sl, :] = out_ref[sl, :] + comm_ref[k, sl, :]
                if k + 1 < N_ROUNDS:
                    rdmas[(k + 1, p)] = make(k + 1, p)
                    rdmas[(k + 1, p)].start()

    return pl.pallas_call(
        body,
        out_shape=jax.ShapeDtypeStruct((m, n), jnp.bfloat16),
        in_specs=[pl.BlockSpec(memory_space=pltpu.VMEM)],
        out_specs=pl.BlockSpec(memory_space=pltpu.VMEM),
        scratch_shapes=[
            pltpu.VMEM((N_ROUNDS, m, n), jnp.bfloat16),
            pltpu.SemaphoreType.DMA((N_ROUNDS, P)),
            pltpu.SemaphoreType.DMA((N_ROUNDS, P)),
        ],
        compiler_params=pltpu.CompilerParams(collective_id=0),
    )(x)


# device time: 14806 ns/iter; 4.2190x vs baseline; 1.2130x over previous
import jax
import jax.numpy as jnp
from jax import lax
from jax.experimental import pallas as pl
from jax.experimental.pallas import tpu as pltpu

N_DEV = 16
ROUND_XOR = (1, 3, 4, 8)
N_ROUNDS = len(ROUND_XOR)
P = 4


def _xor_for(k, p):
    return ROUND_XOR[(k + p) % N_ROUNDS]


def kernel(x):
    m, n = x.shape
    rows = m // P

    def body(x_ref, out_ref, comm_ref, send_sems, recv_sems):
        my = lax.axis_index("i")

        barrier_sem = pltpu.get_barrier_semaphore()
        for xr in ROUND_XOR:
            pl.semaphore_signal(
                barrier_sem,
                inc=1,
                device_id=(lax.bitwise_xor(my, xr),),
                device_id_type=pl.DeviceIdType.MESH,
            )
        pl.semaphore_wait(barrier_sem, N_ROUNDS)

        out_ref[...] = x_ref[...].astype(jnp.bfloat16)

        def make(k, p):
            partner = lax.bitwise_xor(my, _xor_for(k, p))
            return pltpu.make_async_remote_copy(
                src_ref=out_ref.at[pl.ds(p * rows, rows), :],
                dst_ref=comm_ref.at[k, pl.ds(p * rows, rows), :],
                send_sem=send_sems.at[k, p],
                recv_sem=recv_sems.at[k, p],
                device_id=(partner,),
                device_id_type=pl.DeviceIdType.MESH,
            )

        rdmas = {}
        for p in range(P):
            rdmas[(0, p)] = make(0, p)
            rdmas[(0, p)].start()
        for k in range(N_ROUNDS):
            for p in range(P):
                rdmas[(k, p)].wait()
                sl = pl.ds(p * rows, rows)
                out_ref[sl, :] = out_ref[sl, :] + comm_ref[k, sl, :]
                if k + 1 < N_ROUNDS:
                    rdmas[(k + 1, p)] = make(k + 1, p)
                    rdmas[(k + 1, p)].start()

    return pl.pallas_call(
        body,
        out_shape=jax.ShapeDtypeStruct((m, n), jnp.bfloat16),
        in_specs=[pl.BlockSpec(memory_space=pltpu.VMEM)],
        out_specs=pl.BlockSpec(memory_space=pltpu.VMEM),
        scratch_shapes=[
            pltpu.VMEM((N_ROUNDS, m, n), jnp.bfloat16),
            pltpu.SemaphoreType.DMA((N_ROUNDS, P)),
            pltpu.SemaphoreType.DMA((N_ROUNDS, P)),
        ],
        compiler_params=pltpu.CompilerParams(collective_id=0),
    )(x)
